# baseline (speedup 1.0000x reference)
import functools

import numpy as np
import jax
import jax.numpy as jnp
from jax import lax
from jax.experimental import pallas as pl
from jax.experimental.pallas import tpu as pltpu
from jax.experimental.pallas import tpu_sc as plsc

_VOCAB = 1000000
_EMBED = 64
_WINDOW = 2048
_BATCH = 16

_NC = 2
_NS = 16
_L = 16

_BG = _BATCH // 2
_G = _WINDOW // _NS
_SCALE = 8.0


def _pos_encoding_np(length, depth):
    d = depth / 2
    positions = np.arange(length)[:, np.newaxis]
    depths = np.arange(d)[np.newaxis, :] / d
    angle_rates = 1 / 10000 ** depths
    angle_rads = positions * angle_rates
    return np.concatenate(
        [np.sin(angle_rads), np.cos(angle_rads)], axis=-1
    ).astype(np.float32)


def _sc_body(x_hbm, w2_hbm, pe_hbm, out_hbm, idx_v, idx2_v, pe_v, rows_v,
             out_v, sem):
    cid = lax.axis_index("c")
    sid = lax.axis_index("s")
    wid = sid * _NC + cid
    bg = wid // _NS
    tc = wid % _NS
    t0 = tc * _G

    pltpu.sync_copy(x_hbm.at[pl.ds(bg * _BG, _BG), pl.ds(t0, _G)], idx_v)
    pltpu.sync_copy(pe_hbm.at[pl.ds(t0, _G)], pe_v)

    for b in range(_BG):
        for k in range(_G // _L):
            sl = pl.ds(k * _L, _L)
            idx2_v[sl] = lax.shift_right_logical(idx_v[b, sl], 1)
        pltpu.async_copy(w2_hbm.at[idx2_v], rows_v, sem).wait()

        def fma_group(g, carry):
            base = g * _L
            pvec = (idx_v[b, pl.ds(base, _L)] & 1) * _EMBED
            for j in range(_L):
                r = base + j
                p = pvec[j]
                for q in range(_EMBED // _L):
                    out_v[r, pl.ds(q * _L, _L)] = (
                        rows_v[r, pl.ds(p + q * _L, _L)] * _SCALE
                        + pe_v[r, pl.ds(q * _L, _L)]
                    )
            return carry

        lax.fori_loop(0, _G // _L, fma_group, 0)

        row0 = (bg * _BG + b) * _WINDOW + t0
        pltpu.sync_copy(out_v, out_hbm.at[pl.ds(row0, _G)])


def kernel(x, W):
    pe = jnp.asarray(_pos_encoding_np(_WINDOW, _EMBED))
    w2 = W.reshape(_VOCAB // 2, 2 * _EMBED)
    mesh = plsc.VectorSubcoreMesh(core_axis_name="c", subcore_axis_name="s")
    run = functools.partial(
        pl.kernel,
        mesh=mesh,
        out_type=jax.ShapeDtypeStruct((_BATCH * _WINDOW, _EMBED), jnp.float32),
        scratch_types=[
            pltpu.VMEM((_BG, _G), jnp.int32),
            pltpu.VMEM((_G,), jnp.int32),
            pltpu.VMEM((_G, _EMBED), jnp.float32),
            pltpu.VMEM((_G, 2 * _EMBED), jnp.float32),
            pltpu.VMEM((_G, _EMBED), jnp.float32),
            pltpu.SemaphoreType.DMA,
        ],
    )(_sc_body)
    out = run(x, w2, pe)
    return out.reshape(_BATCH, _WINDOW, _EMBED)


# speedup vs baseline: 1.4747x; 1.4747x over previous
import functools

import numpy as np
import jax
import jax.numpy as jnp
from jax import lax
from jax.experimental import pallas as pl
from jax.experimental.pallas import tpu as pltpu
from jax.experimental.pallas import tpu_sc as plsc

_VOCAB = 1000000
_EMBED = 64
_WINDOW = 2048
_BATCH = 16

_NC = 2
_NS = 16
_L = 16

_BG = _BATCH // 2
_G = _WINDOW // _NS
_SCALE = 8.0


def _pos_encoding_np(length, depth):
    d = depth / 2
    positions = np.arange(length)[:, np.newaxis]
    depths = np.arange(d)[np.newaxis, :] / d
    angle_rates = 1 / 10000 ** depths
    angle_rads = positions * angle_rates
    return np.concatenate(
        [np.sin(angle_rads), np.cos(angle_rads)], axis=-1
    ).astype(np.float32)


def _sc_body(x_hbm, w_hbm, pe_hbm, out_hbm, idx_v, pe_v, rows_v, out_v, sem):
    cid = lax.axis_index("c")
    sid = lax.axis_index("s")
    wid = sid * _NC + cid
    bg = wid // _NS
    tc = wid % _NS
    t0 = tc * _G

    pltpu.sync_copy(x_hbm.at[pl.ds(bg * _BG, _BG), pl.ds(t0, _G)], idx_v)
    pltpu.sync_copy(pe_hbm.at[pl.ds(t0, _G)], pe_v)

    _H = _G // 2

    for b in range(_BG):
        for h in range(2):

            def issue_group(g, carry, _h=h):
                vec = idx_v[b, pl.ds(_h * _H + g * _L, _L)]
                base = lax.shift_left(lax.shift_right_logical(vec, 3), 3)
                for j in range(_L):
                    i8 = pl.multiple_of(base[j], 8)
                    pltpu.async_copy(
                        w_hbm.at[pl.ds(i8, 8)],
                        rows_v.at[pl.ds((g * _L + j) * 8, 8)],
                        sem,
                    )
                return carry

            lax.fori_loop(0, _H // _L, issue_group, 0)
            pltpu.make_async_copy(
                w_hbm.at[pl.ds(0, _H * 8)], rows_v, sem
            ).wait()

            def fma_group(g, carry, _h=h):
                sub = idx_v[b, pl.ds(_h * _H + g * _L, _L)] & 7
                for j in range(_L):
                    s = g * _L + j
                    r = s * 8 + sub[j]
                    for q in range(_EMBED // _L):
                        sl = pl.ds(q * _L, _L)
                        out_v[_h * _H + s, sl] = (
                            rows_v[r, sl] * _SCALE + pe_v[_h * _H + s, sl]
                        )
                return carry

            lax.fori_loop(0, _H // _L, fma_group, 0)

        row0 = (bg * _BG + b) * _WINDOW + t0
        pltpu.sync_copy(out_v, out_hbm.at[pl.ds(row0, _G)])


def kernel(x, W):
    pe = jnp.asarray(_pos_encoding_np(_WINDOW, _EMBED))
    mesh = plsc.VectorSubcoreMesh(core_axis_name="c", subcore_axis_name="s")
    run = functools.partial(
        pl.kernel,
        mesh=mesh,
        out_type=jax.ShapeDtypeStruct((_BATCH * _WINDOW, _EMBED), jnp.float32),
        scratch_types=[
            pltpu.VMEM((_BG, _G), jnp.int32),
            pltpu.VMEM((_G, _EMBED), jnp.float32),
            pltpu.VMEM((_G * 4, _EMBED), jnp.float32),
            pltpu.VMEM((_G, _EMBED), jnp.float32),
            pltpu.SemaphoreType.DMA,
        ],
    )(_sc_body)
    out = run(x, W, pe)
    return out.reshape(_BATCH, _WINDOW, _EMBED)


# speedup vs baseline: 1.9854x; 1.3463x over previous
import functools

import numpy as np
import jax
import jax.numpy as jnp
from jax import lax
from jax.experimental import pallas as pl
from jax.experimental.pallas import tpu as pltpu
from jax.experimental.pallas import tpu_sc as plsc

_VOCAB = 1000000
_EMBED = 64
_WINDOW = 2048
_BATCH = 16

_NC = 2
_NS = 16
_L = 16

_BG = _BATCH // 2
_G = _WINDOW // _NS
_SCALE = 8.0


def _pos_encoding_np(length, depth):
    d = depth / 2
    positions = np.arange(length)[:, np.newaxis]
    depths = np.arange(d)[np.newaxis, :] / d
    angle_rates = 1 / 10000 ** depths
    angle_rads = positions * angle_rates
    return np.concatenate(
        [np.sin(angle_rads), np.cos(angle_rads)], axis=-1
    ).astype(np.float32)


def _sc_body(x_hbm, w3_hbm, pe_hbm, out_hbm, idx_v, pe_v, rows_v, out_v, sem):
    cid = lax.axis_index("c")
    sid = lax.axis_index("s")
    wid = sid * _NC + cid
    bg = wid // _NS
    tc = wid % _NS
    t0 = tc * _G

    pltpu.sync_copy(x_hbm.at[pl.ds(bg * _BG, _BG), pl.ds(t0, _G)], idx_v)
    pltpu.sync_copy(pe_hbm.at[pl.ds(t0, _G)], pe_v)

    _H = _G // 2

    for b in range(_BG):
        for h in range(2):

            def issue_group(g, carry, _h=h):
                grp = lax.shift_right_logical(
                    idx_v[b, pl.ds(_h * _H + g * _L, _L)], 3
                )
                for j in range(_L):
                    pltpu.async_copy(
                        w3_hbm.at[grp[j]],
                        rows_v.at[pl.ds((g * _L + j) * 8, 8)],
                        sem,
                    )
                return carry

            lax.fori_loop(0, _H // _L, issue_group, 0)
            pltpu.make_async_copy(
                out_hbm.at[pl.ds(0, _H * 8)], rows_v, sem
            ).wait()

            def fma_group(g, carry, _h=h):
                sub = idx_v[b, pl.ds(_h * _H + g * _L, _L)] & 7
                for j in range(_L):
                    s = g * _L + j
                    r = s * 8 + sub[j]
                    for q in range(_EMBED // _L):
                        sl = pl.ds(q * _L, _L)
                        out_v[_h * _H + s, sl] = (
                            rows_v[r, sl] * _SCALE + pe_v[_h * _H + s, sl]
                        )
                return carry

            lax.fori_loop(0, _H // _L, fma_group, 0)

        row0 = (bg * _BG + b) * _WINDOW + t0
        pltpu.sync_copy(out_v, out_hbm.at[pl.ds(row0, _G)])


def kernel(x, W):
    pe = jnp.asarray(_pos_encoding_np(_WINDOW, _EMBED))
    w3 = W.reshape(_VOCAB // 8, 8, _EMBED)
    mesh = plsc.VectorSubcoreMesh(core_axis_name="c", subcore_axis_name="s")
    run = functools.partial(
        pl.kernel,
        mesh=mesh,
        out_type=jax.ShapeDtypeStruct((_BATCH * _WINDOW, _EMBED), jnp.float32),
        scratch_types=[
            pltpu.VMEM((_BG, _G), jnp.int32),
            pltpu.VMEM((_G, _EMBED), jnp.float32),
            pltpu.VMEM((_G * 4, _EMBED), jnp.float32),
            pltpu.VMEM((_G, _EMBED), jnp.float32),
            pltpu.SemaphoreType.DMA,
        ],
    )(_sc_body)
    out = run(x, w3, pe)
    return out.reshape(_BATCH, _WINDOW, _EMBED)
